# initial kernel scaffold (unmeasured)
import jax
import jax.numpy as jnp
from jax import lax
from jax.experimental import pallas as pl
from jax.experimental.pallas import tpu as pltpu

N_DEV = 4
N_EXP = 8
CAP = 160


def _moe_a2a_pallas(send_buf, W1, W2):
    _, _, D = send_buf.shape
    F = W1.shape[2]
    del F

    def body(send_ref, w1_ref, w2_ref, out_ref,
             recv_ref, y_ref, ret_ref,
             disp_send_sems, disp_recv_sems, ret_send_sems, ret_recv_sems):
        me = lax.axis_index("i")

        bar = pltpu.get_barrier_semaphore()
        for off in (1, 2, 3):
            pl.semaphore_signal(
                bar, inc=1,
                device_id=((me + off) % N_DEV,),
                device_id_type=pl.DeviceIdType.MESH,
            )
        pl.semaphore_wait(bar, N_DEV - 1)

        recv_ref[pl.ds(me, 1)] = send_ref[pl.ds(2 * me, 2)][None]
        disp = []
        for off in (1, 2, 3):
            d = (me + off) % N_DEV
            r = pltpu.make_async_remote_copy(
                src_ref=send_ref.at[pl.ds(2 * d, 2)],
                dst_ref=recv_ref.at[me],
                send_sem=disp_send_sems.at[d],
                recv_sem=disp_recv_sems.at[me],
                device_id=(d,),
                device_id_type=pl.DeviceIdType.MESH,
            )
            r.start()
            disp.append(r)
        for off in (1, 2, 3):
            s = (me + off) % N_DEV
            pltpu.make_async_remote_copy(
                src_ref=send_ref.at[pl.ds(0, 2)],
                dst_ref=recv_ref.at[s],
                send_sem=disp_send_sems.at[s],
                recv_sem=disp_recv_sems.at[s],
                device_id=(s,),
                device_id_type=pl.DeviceIdType.MESH,
            ).wait_recv()

        for k in range(2):
            a = recv_ref[:, k, :, :].reshape(N_DEV * CAP, D)
            h = jnp.maximum(
                jnp.dot(a, w1_ref[k], preferred_element_type=jnp.float32), 0.0)
            y = jnp.dot(h, w2_ref[k], preferred_element_type=jnp.float32)
            y_ref[:, k, :, :] = y.reshape(N_DEV, CAP, D)

        ret_ref[pl.ds(2 * me, 2)] = y_ref[pl.ds(me, 1)].reshape(2, CAP, D)
        rets = []
        for off in (1, 2, 3):
            s = (me + off) % N_DEV
            r = pltpu.make_async_remote_copy(
                src_ref=y_ref.at[s],
                dst_ref=ret_ref.at[pl.ds(2 * me, 2)],
                send_sem=ret_send_sems.at[s],
                recv_sem=ret_recv_sems.at[me],
                device_id=(s,),
                device_id_type=pl.DeviceIdType.MESH,
            )
            r.start()
            rets.append(r)
        for off in (1, 2, 3):
            s = (me + off) % N_DEV
            pltpu.make_async_remote_copy(
                src_ref=y_ref.at[s],
                dst_ref=ret_ref.at[pl.ds(2 * s, 2)],
                send_sem=ret_send_sems.at[s],
                recv_sem=ret_recv_sems.at[s],
                device_id=(s,),
                device_id_type=pl.DeviceIdType.MESH,
            ).wait_recv()

        out_ref[...] = ret_ref[...]
        for r in disp + rets:
            r.wait_send()

    return pl.pallas_call(
        body,
        out_shape=jax.ShapeDtypeStruct((N_EXP, CAP, D), jnp.float32),
        in_specs=[pl.BlockSpec(memory_space=pltpu.VMEM)] * 3,
        out_specs=pl.BlockSpec(memory_space=pltpu.VMEM),
        scratch_shapes=[
            pltpu.VMEM((N_DEV, 2, CAP, D), jnp.float32),
            pltpu.VMEM((N_DEV, 2, CAP, D), jnp.float32),
            pltpu.VMEM((N_EXP, CAP, D), jnp.float32),
            pltpu.SemaphoreType.DMA((N_DEV,)),
            pltpu.SemaphoreType.DMA((N_DEV,)),
            pltpu.SemaphoreType.DMA((N_DEV,)),
            pltpu.SemaphoreType.DMA((N_DEV,)),
        ],
        compiler_params=pltpu.CompilerParams(collective_id=0),
    )(send_buf, W1, W2)


def kernel(x, assign, W1, W2):
    T, D = x.shape
    e = assign.astype(jnp.int32)

    sort_idx = jnp.argsort(e, stable=True).astype(jnp.int32)
    sorted_e = e[sort_idx]
    group_start = jnp.searchsorted(sorted_e, jnp.arange(N_EXP, dtype=jnp.int32))
    rank_sorted = jnp.arange(T, dtype=jnp.int32) - group_start[sorted_e].astype(jnp.int32)
    slots_sorted = sorted_e * CAP + rank_sorted

    inv = jnp.full((N_EXP * CAP,), T, jnp.int32).at[slots_sorted].set(sort_idx)
    x_pad = jnp.concatenate([x, jnp.zeros((1, D), x.dtype)], axis=0)
    send_buf = x_pad[inv].reshape(N_EXP, CAP, D)

    ret = _moe_a2a_pallas(send_buf, W1, W2)

    slot_of_token = jnp.zeros((T,), jnp.int32).at[sort_idx].set(slots_sorted)
    return ret.reshape(N_EXP * CAP, D)[slot_of_token]


# baseline (device time: 398093 ns/iter reference)
import jax
import jax.numpy as jnp
from jax import lax
from jax.experimental import pallas as pl
from jax.experimental.pallas import tpu as pltpu

N_DEV = 4
N_EXP = 8
CAP = 160


def _moe_a2a_pallas(send_buf, W1, W2):
    _, _, D = send_buf.shape
    F = W1.shape[2]
    del F

    def body(send_ref, w1_ref, w2_ref, out_ref,
             recv_ref, y_ref, ret_ref,
             disp_send_sems, disp_recv_sems, ret_send_sems, ret_recv_sems):
        me = lax.axis_index("i")

        bar = pltpu.get_barrier_semaphore()
        for off in (1, 2, 3):
            pl.semaphore_signal(
                bar, inc=1,
                device_id=((me + off) % N_DEV,),
                device_id_type=pl.DeviceIdType.MESH,
            )
        pl.semaphore_wait(bar, N_DEV - 1)

        recv_ref[pl.ds(me, 1)] = send_ref[pl.ds(2 * me, 2)][None]
        disp = []
        for off in (1, 2, 3):
            d = (me + off) % N_DEV
            r = pltpu.make_async_remote_copy(
                src_ref=send_ref.at[pl.ds(2 * d, 2)],
                dst_ref=recv_ref.at[me],
                send_sem=disp_send_sems.at[d],
                recv_sem=disp_recv_sems.at[me],
                device_id=(d,),
                device_id_type=pl.DeviceIdType.MESH,
            )
            r.start()
            disp.append(r)
        for off in (1, 2, 3):
            s = (me + off) % N_DEV
            pltpu.make_async_remote_copy(
                src_ref=send_ref.at[pl.ds(0, 2)],
                dst_ref=recv_ref.at[s],
                send_sem=disp_send_sems.at[s],
                recv_sem=disp_recv_sems.at[s],
                device_id=(s,),
                device_id_type=pl.DeviceIdType.MESH,
            ).wait_recv()

        for k in range(2):
            a = recv_ref[:, k, :, :].reshape(N_DEV * CAP, D)
            h = jnp.maximum(
                jnp.dot(a, w1_ref[k], preferred_element_type=jnp.float32), 0.0)
            y = jnp.dot(h, w2_ref[k], preferred_element_type=jnp.float32)
            y_ref[:, k, :, :] = y.reshape(N_DEV, CAP, D)

        ret_ref[pl.ds(2 * me, 2)] = y_ref[pl.ds(me, 1)].reshape(2, CAP, D)
        rets = []
        for off in (1, 2, 3):
            s = (me + off) % N_DEV
            r = pltpu.make_async_remote_copy(
                src_ref=y_ref.at[s],
                dst_ref=ret_ref.at[pl.ds(2 * me, 2)],
                send_sem=ret_send_sems.at[s],
                recv_sem=ret_recv_sems.at[me],
                device_id=(s,),
                device_id_type=pl.DeviceIdType.MESH,
            )
            r.start()
            rets.append(r)
        for off in (1, 2, 3):
            s = (me + off) % N_DEV
            pltpu.make_async_remote_copy(
                src_ref=y_ref.at[s],
                dst_ref=ret_ref.at[pl.ds(2 * s, 2)],
                send_sem=ret_send_sems.at[s],
                recv_sem=ret_recv_sems.at[s],
                device_id=(s,),
                device_id_type=pl.DeviceIdType.MESH,
            ).wait_recv()

        out_ref[...] = ret_ref[...]
        for r in disp + rets:
            r.wait_send()

    return pl.pallas_call(
        body,
        out_shape=jax.ShapeDtypeStruct((N_EXP, CAP, D), jnp.float32),
        in_specs=[pl.BlockSpec(memory_space=pltpu.VMEM)] * 3,
        out_specs=pl.BlockSpec(memory_space=pltpu.VMEM),
        scratch_shapes=[
            pltpu.VMEM((N_DEV, 2, CAP, D), jnp.float32),
            pltpu.VMEM((N_DEV, 2, CAP, D), jnp.float32),
            pltpu.VMEM((N_EXP, CAP, D), jnp.float32),
            pltpu.SemaphoreType.DMA((N_DEV,)),
            pltpu.SemaphoreType.DMA((N_DEV,)),
            pltpu.SemaphoreType.DMA((N_DEV,)),
            pltpu.SemaphoreType.DMA((N_DEV,)),
        ],
        compiler_params=pltpu.CompilerParams(
            collective_id=0, vmem_limit_bytes=100 * 1024 * 1024),
    )(send_buf, W1, W2)


def kernel(x, assign, W1, W2):
    T, D = x.shape
    e = assign.astype(jnp.int32)

    sort_idx = jnp.argsort(e, stable=True).astype(jnp.int32)
    sorted_e = e[sort_idx]
    group_start = jnp.searchsorted(sorted_e, jnp.arange(N_EXP, dtype=jnp.int32))
    rank_sorted = jnp.arange(T, dtype=jnp.int32) - group_start[sorted_e].astype(jnp.int32)
    slots_sorted = sorted_e * CAP + rank_sorted

    inv = jnp.full((N_EXP * CAP,), T, jnp.int32).at[slots_sorted].set(sort_idx)
    x_pad = jnp.concatenate([x, jnp.zeros((1, D), x.dtype)], axis=0)
    send_buf = x_pad[inv].reshape(N_EXP, CAP, D)

    ret = _moe_a2a_pallas(send_buf, W1, W2)

    slot_of_token = jnp.zeros((T,), jnp.int32).at[sort_idx].set(slots_sorted)
    return ret.reshape(N_EXP * CAP, D)[slot_of_token]


# device time: 103121 ns/iter; 3.8604x vs baseline; 3.8604x over previous
import jax
import jax.numpy as jnp
from jax import lax
from jax.experimental import pallas as pl
from jax.experimental.pallas import tpu as pltpu

N_DEV = 4
N_EXP = 8
CAP = 160
S = N_EXP * CAP


def _moe_a2a_pallas(x, slot2d, W1, W2):
    T, D = x.shape

    def body(x_ref, slot_ref, w1_ref, w2_ref, out_ref,
             send_ref, recv_ref, y_ref,
             disp_send_sems, disp_recv_sems, ret_send_sems, ret_recv_sems):
        ret_ref = send_ref
        me = lax.axis_index("i")

        bar = pltpu.get_barrier_semaphore()
        for off in (1, 2, 3):
            pl.semaphore_signal(
                bar, inc=1,
                device_id=((me + off) % N_DEV,),
                device_id_type=pl.DeviceIdType.MESH,
            )
        pl.semaphore_wait(bar, N_DEV - 1)

        def placement():
            s_iota = lax.broadcasted_iota(jnp.int32, (S, T), 0)
            return (slot_ref[0, :][None, :] == s_iota).astype(jnp.float32)

        send = jnp.dot(placement(), x_ref[...],
                       preferred_element_type=jnp.float32)
        send_ref[...] = send.reshape(N_EXP, CAP, D)

        recv_ref[pl.ds(me, 1)] = send_ref[pl.ds(2 * me, 2)][None]
        disp = []
        for off in (1, 2, 3):
            d = (me + off) % N_DEV
            r = pltpu.make_async_remote_copy(
                src_ref=send_ref.at[pl.ds(2 * d, 2)],
                dst_ref=recv_ref.at[me],
                send_sem=disp_send_sems.at[d],
                recv_sem=disp_recv_sems.at[me],
                device_id=(d,),
                device_id_type=pl.DeviceIdType.MESH,
            )
            r.start()
            disp.append(r)
        for off in (1, 2, 3):
            s = (me + off) % N_DEV
            pltpu.make_async_remote_copy(
                src_ref=send_ref.at[pl.ds(0, 2)],
                dst_ref=recv_ref.at[s],
                send_sem=disp_send_sems.at[s],
                recv_sem=disp_recv_sems.at[s],
                device_id=(s,),
                device_id_type=pl.DeviceIdType.MESH,
            ).wait_recv()

        for k in range(2):
            a = recv_ref[:, k, :, :].reshape(N_DEV * CAP, D)
            h = jnp.maximum(
                jnp.dot(a, w1_ref[k], preferred_element_type=jnp.float32), 0.0)
            y = jnp.dot(h, w2_ref[k], preferred_element_type=jnp.float32)
            y_ref[:, k, :, :] = y.reshape(N_DEV, CAP, D)

        ret_ref[pl.ds(2 * me, 2)] = y_ref[pl.ds(me, 1)].reshape(2, CAP, D)
        rets = []
        for off in (1, 2, 3):
            s = (me + off) % N_DEV
            r = pltpu.make_async_remote_copy(
                src_ref=y_ref.at[s],
                dst_ref=ret_ref.at[pl.ds(2 * me, 2)],
                send_sem=ret_send_sems.at[s],
                recv_sem=ret_recv_sems.at[me],
                device_id=(s,),
                device_id_type=pl.DeviceIdType.MESH,
            )
            r.start()
            rets.append(r)
        for off in (1, 2, 3):
            s = (me + off) % N_DEV
            pltpu.make_async_remote_copy(
                src_ref=y_ref.at[s],
                dst_ref=ret_ref.at[pl.ds(2 * s, 2)],
                send_sem=ret_send_sems.at[s],
                recv_sem=ret_recv_sems.at[s],
                device_id=(s,),
                device_id_type=pl.DeviceIdType.MESH,
            ).wait_recv()

        out_ref[...] = lax.dot_general(
            placement(), ret_ref[...].reshape(S, D),
            (((0,), (0,)), ((), ())),
            preferred_element_type=jnp.float32,
        )
        for r in disp + rets:
            r.wait_send()

    return pl.pallas_call(
        body,
        out_shape=jax.ShapeDtypeStruct((T, D), jnp.float32),
        in_specs=[pl.BlockSpec(memory_space=pltpu.VMEM)] * 4,
        out_specs=pl.BlockSpec(memory_space=pltpu.VMEM),
        scratch_shapes=[
            pltpu.VMEM((N_EXP, CAP, D), jnp.float32),
            pltpu.VMEM((N_DEV, 2, CAP, D), jnp.float32),
            pltpu.VMEM((N_DEV, 2, CAP, D), jnp.float32),
            pltpu.SemaphoreType.DMA((N_DEV,)),
            pltpu.SemaphoreType.DMA((N_DEV,)),
            pltpu.SemaphoreType.DMA((N_DEV,)),
            pltpu.SemaphoreType.DMA((N_DEV,)),
        ],
        compiler_params=pltpu.CompilerParams(
            collective_id=0, vmem_limit_bytes=110 * 1024 * 1024),
    )(x, slot2d, W1, W2)


def kernel(x, assign, W1, W2):
    T, _ = x.shape
    e = assign.astype(jnp.int32)

    oh = (e[:, None] == jnp.arange(N_EXP, dtype=jnp.int32)[None, :]).astype(
        jnp.int32)
    csum = jnp.cumsum(oh, axis=0)
    rank = jnp.sum(oh * csum, axis=1) - 1
    slot = jnp.where(rank < CAP, e * CAP + rank, S)

    return _moe_a2a_pallas(x, slot.reshape(1, T), W1, W2)


# device time: 92986 ns/iter; 4.2812x vs baseline; 1.1090x over previous
import jax
import jax.numpy as jnp
from jax import lax
from jax.experimental import pallas as pl
from jax.experimental.pallas import tpu as pltpu

N_DEV = 4
N_EXP = 8
CAP = 160
S = N_EXP * CAP


def _moe_a2a_pallas(x, slot2d, W1, W2):
    T, D = x.shape

    def body(x_ref, slot_ref, w1_ref, w2_ref, out_ref,
             send_ref, recv_ref,
             disp_send_sems, disp_recv_sems, ret_send_sems, ret_recv_sems):
        ret_ref = send_ref
        y_ref = recv_ref
        me = lax.axis_index("i")

        bar = pltpu.get_barrier_semaphore()
        for off in (1, 2, 3):
            pl.semaphore_signal(
                bar, inc=1,
                device_id=((me + off) % N_DEV,),
                device_id_type=pl.DeviceIdType.MESH,
            )
        pl.semaphore_wait(bar, N_DEV - 1)

        def p_chunk(d):
            s_iota = (lax.broadcasted_iota(jnp.int32, (2 * CAP, T), 0)
                      + 2 * d * CAP)
            return (slot_ref[0, :][None, :] == s_iota).astype(jnp.float32)

        disp = []
        for off in (1, 2, 3):
            d = (me + off) % N_DEV
            chunk = jnp.dot(p_chunk(d), x_ref[...],
                            preferred_element_type=jnp.float32)
            send_ref[pl.ds(2 * d, 2)] = chunk.reshape(2, CAP, D)
            r = pltpu.make_async_remote_copy(
                src_ref=send_ref.at[pl.ds(2 * d, 2)],
                dst_ref=recv_ref.at[me],
                send_sem=disp_send_sems.at[d],
                recv_sem=disp_recv_sems.at[me],
                device_id=(d,),
                device_id_type=pl.DeviceIdType.MESH,
            )
            r.start()
            disp.append(r)

        own = jnp.dot(p_chunk(me), x_ref[...],
                      preferred_element_type=jnp.float32)
        recv_ref[pl.ds(me, 1)] = own.reshape(1, 2, CAP, D)

        def mlp(s):
            for k in range(2):
                a = recv_ref[pl.ds(s, 1), k, :, :].reshape(CAP, D)
                h = jnp.maximum(
                    jnp.dot(a, w1_ref[k], preferred_element_type=jnp.float32),
                    0.0)
                y = jnp.dot(h, w2_ref[k], preferred_element_type=jnp.float32)
                y_ref[pl.ds(s, 1), k, :, :] = y.reshape(1, CAP, D)

        mlp(me)
        ret_ref[pl.ds(2 * me, 2)] = y_ref[pl.ds(me, 1)].reshape(2, CAP, D)

        rets = []
        for off in (3, 2, 1):
            s = (me + off) % N_DEV
            pltpu.make_async_remote_copy(
                src_ref=send_ref.at[pl.ds(0, 2)],
                dst_ref=recv_ref.at[s],
                send_sem=disp_send_sems.at[s],
                recv_sem=disp_recv_sems.at[s],
                device_id=(s,),
                device_id_type=pl.DeviceIdType.MESH,
            ).wait_recv()
            mlp(s)
            r = pltpu.make_async_remote_copy(
                src_ref=y_ref.at[s],
                dst_ref=ret_ref.at[pl.ds(2 * me, 2)],
                send_sem=ret_send_sems.at[s],
                recv_sem=ret_recv_sems.at[me],
                device_id=(s,),
                device_id_type=pl.DeviceIdType.MESH,
            )
            r.start()
            rets.append(r)

        out_ref[...] = lax.dot_general(
            p_chunk(me), ret_ref[pl.ds(2 * me, 2)].reshape(2 * CAP, D),
            (((0,), (0,)), ((), ())), preferred_element_type=jnp.float32)
        for off in (1, 2, 3):
            s = (me + off) % N_DEV
            pltpu.make_async_remote_copy(
                src_ref=y_ref.at[s],
                dst_ref=ret_ref.at[pl.ds(2 * s, 2)],
                send_sem=ret_send_sems.at[s],
                recv_sem=ret_recv_sems.at[s],
                device_id=(s,),
                device_id_type=pl.DeviceIdType.MESH,
            ).wait_recv()
            out_ref[...] = out_ref[...] + lax.dot_general(
                p_chunk(s), ret_ref[pl.ds(2 * s, 2)].reshape(2 * CAP, D),
                (((0,), (0,)), ((), ())), preferred_element_type=jnp.float32)

        for r in disp + rets:
            r.wait_send()

    return pl.pallas_call(
        body,
        out_shape=jax.ShapeDtypeStruct((T, D), jnp.float32),
        in_specs=[pl.BlockSpec(memory_space=pltpu.VMEM)] * 4,
        out_specs=pl.BlockSpec(memory_space=pltpu.VMEM),
        scratch_shapes=[
            pltpu.VMEM((N_EXP, CAP, D), jnp.float32),
            pltpu.VMEM((N_DEV, 2, CAP, D), jnp.float32),
            pltpu.SemaphoreType.DMA((N_DEV,)),
            pltpu.SemaphoreType.DMA((N_DEV,)),
            pltpu.SemaphoreType.DMA((N_DEV,)),
            pltpu.SemaphoreType.DMA((N_DEV,)),
        ],
        compiler_params=pltpu.CompilerParams(
            collective_id=0, vmem_limit_bytes=110 * 1024 * 1024),
    )(x, slot2d, W1, W2)


def kernel(x, assign, W1, W2):
    T, _ = x.shape
    e = assign.astype(jnp.int32)

    oh = (e[:, None] == jnp.arange(N_EXP, dtype=jnp.int32)[None, :]).astype(
        jnp.int32)
    csum = jnp.cumsum(oh, axis=0)
    rank = jnp.sum(oh * csum, axis=1) - 1
    slot = jnp.where(rank < CAP, e * CAP + rank, S)

    return _moe_a2a_pallas(x, slot.reshape(1, T), W1, W2)


# device time: 79718 ns/iter; 4.9938x vs baseline; 1.1664x over previous
import jax
import jax.numpy as jnp
from jax import lax
from jax.experimental import pallas as pl
from jax.experimental.pallas import tpu as pltpu

N_DEV = 4
N_EXP = 8
CAP = 160
S = N_EXP * CAP


def _moe_a2a_pallas(x, slot2d, W1, W2):
    T, D = x.shape

    def body(x_ref, slot_ref, w1_ref, w2_ref, out_ref,
             send_ref, recv_ref,
             disp_send_sems, disp_recv_sems, ret_send_sems, ret_recv_sems):
        ret_ref = send_ref
        y_ref = recv_ref
        me = lax.axis_index("i")

        bar = pltpu.get_barrier_semaphore()
        for off in (1, 2, 3):
            pl.semaphore_signal(
                bar, inc=1,
                device_id=((me + off) % N_DEV,),
                device_id_type=pl.DeviceIdType.MESH,
            )
        pl.semaphore_wait(bar, N_DEV - 1)

        def p_chunk(d):
            s_iota = (lax.broadcasted_iota(jnp.int32, (2 * CAP, T), 0)
                      + 2 * d * CAP)
            return (slot_ref[0, :][None, :] == s_iota).astype(jnp.bfloat16)

        disp = []
        for off in (1, 2, 3):
            d = (me + off) % N_DEV
            chunk = jnp.dot(p_chunk(d), x_ref[...],
                            preferred_element_type=jnp.float32)
            send_ref[pl.ds(2 * d, 2)] = (
                chunk.astype(jnp.bfloat16).reshape(2, CAP, D))
            r = pltpu.make_async_remote_copy(
                src_ref=send_ref.at[pl.ds(2 * d, 2)],
                dst_ref=recv_ref.at[me],
                send_sem=disp_send_sems.at[d],
                recv_sem=disp_recv_sems.at[me],
                device_id=(d,),
                device_id_type=pl.DeviceIdType.MESH,
            )
            r.start()
            disp.append(r)

        own = jnp.dot(p_chunk(me), x_ref[...],
                      preferred_element_type=jnp.float32)
        recv_ref[pl.ds(me, 1)] = (
            own.astype(jnp.bfloat16).reshape(1, 2, CAP, D))

        def mlp(s):
            for k in range(2):
                a = recv_ref[pl.ds(s, 1), k, :, :].reshape(CAP, D)
                h = jnp.maximum(
                    jnp.dot(a, w1_ref[k], preferred_element_type=jnp.float32),
                    0.0)
                y = jnp.dot(h.astype(jnp.bfloat16), w2_ref[k],
                            preferred_element_type=jnp.float32)
                y_ref[pl.ds(s, 1), k, :, :] = (
                    y.astype(jnp.bfloat16).reshape(1, CAP, D))

        mlp(me)
        ret_ref[pl.ds(2 * me, 2)] = y_ref[pl.ds(me, 1)].reshape(2, CAP, D)

        rets = []
        for off in (3, 2, 1):
            s = (me + off) % N_DEV
            pltpu.make_async_remote_copy(
                src_ref=send_ref.at[pl.ds(0, 2)],
                dst_ref=recv_ref.at[s],
                send_sem=disp_send_sems.at[s],
                recv_sem=disp_recv_sems.at[s],
                device_id=(s,),
                device_id_type=pl.DeviceIdType.MESH,
            ).wait_recv()
            mlp(s)
            r = pltpu.make_async_remote_copy(
                src_ref=y_ref.at[s],
                dst_ref=ret_ref.at[pl.ds(2 * me, 2)],
                send_sem=ret_send_sems.at[s],
                recv_sem=ret_recv_sems.at[me],
                device_id=(s,),
                device_id_type=pl.DeviceIdType.MESH,
            )
            r.start()
            rets.append(r)

        out_ref[...] = lax.dot_general(
            p_chunk(me), ret_ref[pl.ds(2 * me, 2)].reshape(2 * CAP, D),
            (((0,), (0,)), ((), ())), preferred_element_type=jnp.float32)
        for off in (1, 2, 3):
            s = (me + off) % N_DEV
            pltpu.make_async_remote_copy(
                src_ref=y_ref.at[s],
                dst_ref=ret_ref.at[pl.ds(2 * s, 2)],
                send_sem=ret_send_sems.at[s],
                recv_sem=ret_recv_sems.at[s],
                device_id=(s,),
                device_id_type=pl.DeviceIdType.MESH,
            ).wait_recv()
            out_ref[...] = out_ref[...] + lax.dot_general(
                p_chunk(s), ret_ref[pl.ds(2 * s, 2)].reshape(2 * CAP, D),
                (((0,), (0,)), ((), ())), preferred_element_type=jnp.float32)

        for r in disp + rets:
            r.wait_send()

    return pl.pallas_call(
        body,
        out_shape=jax.ShapeDtypeStruct((T, D), jnp.float32),
        in_specs=[pl.BlockSpec(memory_space=pltpu.VMEM)] * 4,
        out_specs=pl.BlockSpec(memory_space=pltpu.VMEM),
        scratch_shapes=[
            pltpu.VMEM((N_EXP, CAP, D), jnp.bfloat16),
            pltpu.VMEM((N_DEV, 2, CAP, D), jnp.bfloat16),
            pltpu.SemaphoreType.DMA((N_DEV,)),
            pltpu.SemaphoreType.DMA((N_DEV,)),
            pltpu.SemaphoreType.DMA((N_DEV,)),
            pltpu.SemaphoreType.DMA((N_DEV,)),
        ],
        compiler_params=pltpu.CompilerParams(
            collective_id=0, vmem_limit_bytes=110 * 1024 * 1024),
    )(x, slot2d, W1, W2)


def kernel(x, assign, W1, W2):
    T, _ = x.shape
    e = assign.astype(jnp.int32)

    oh = (e[:, None] == jnp.arange(N_EXP, dtype=jnp.int32)[None, :]).astype(
        jnp.int32)
    csum = jnp.cumsum(oh, axis=0)
    rank = jnp.sum(oh * csum, axis=1) - 1
    slot = jnp.where(rank < CAP, e * CAP + rank, S)

    return _moe_a2a_pallas(
        x.astype(jnp.bfloat16), slot.reshape(1, T),
        W1.astype(jnp.bfloat16), W2.astype(jnp.bfloat16))


# device time: 54875 ns/iter; 7.2545x vs baseline; 1.4527x over previous
import jax
import jax.numpy as jnp
from jax import lax
from jax.experimental import pallas as pl
from jax.experimental.pallas import tpu as pltpu

N_DEV = 4
N_EXP = 8
CAP = 160
S = N_EXP * CAP


def _moe_a2a_pallas(x, slot2d, W1, W2):
    T, D = x.shape

    def body(x_ref, slot_ref, w1_hbm, w2_hbm, out_ref,
             send_ref, recv_ref, w1_ref, w2_ref,
             disp_send_sems, disp_recv_sems, ret_send_sems, ret_recv_sems,
             w_sems):
        ret_ref = send_ref
        y_ref = recv_ref
        me = lax.axis_index("i")

        w1_copy = pltpu.make_async_copy(w1_hbm, w1_ref, w_sems.at[0])
        w2_copy = pltpu.make_async_copy(w2_hbm, w2_ref, w_sems.at[1])
        w1_copy.start()
        w2_copy.start()

        bar = pltpu.get_barrier_semaphore()
        for off in (1, 2, 3):
            pl.semaphore_signal(
                bar, inc=1,
                device_id=((me + off) % N_DEV,),
                device_id_type=pl.DeviceIdType.MESH,
            )
        pl.semaphore_wait(bar, N_DEV - 1)

        def p_chunk(d):
            s_iota = (lax.broadcasted_iota(jnp.int32, (2 * CAP, T), 0)
                      + 2 * d * CAP)
            return (slot_ref[0, :][None, :] == s_iota).astype(jnp.bfloat16)

        disp = []
        for off in (1, 2, 3):
            d = (me + off) % N_DEV
            chunk = jnp.dot(p_chunk(d), x_ref[...],
                            preferred_element_type=jnp.float32)
            send_ref[pl.ds(2 * d, 2)] = (
                chunk.astype(jnp.bfloat16).reshape(2, CAP, D))
            r = pltpu.make_async_remote_copy(
                src_ref=send_ref.at[pl.ds(2 * d, 2)],
                dst_ref=recv_ref.at[me],
                send_sem=disp_send_sems.at[d],
                recv_sem=disp_recv_sems.at[me],
                device_id=(d,),
                device_id_type=pl.DeviceIdType.MESH,
            )
            r.start()
            disp.append(r)

        own = jnp.dot(p_chunk(me), x_ref[...],
                      preferred_element_type=jnp.float32)
        recv_ref[pl.ds(me, 1)] = (
            own.astype(jnp.bfloat16).reshape(1, 2, CAP, D))

        def mlp(s):
            for k in range(2):
                a = recv_ref[pl.ds(s, 1), k, :, :].reshape(CAP, D)
                h = jnp.maximum(
                    jnp.dot(a, w1_ref[k], preferred_element_type=jnp.float32),
                    0.0)
                y = jnp.dot(h.astype(jnp.bfloat16), w2_ref[k],
                            preferred_element_type=jnp.float32)
                y_ref[pl.ds(s, 1), k, :, :] = (
                    y.astype(jnp.bfloat16).reshape(1, CAP, D))

        w1_copy.wait()
        w2_copy.wait()
        mlp(me)
        ret_ref[pl.ds(2 * me, 2)] = y_ref[pl.ds(me, 1)].reshape(2, CAP, D)

        rets = []
        for off in (3, 2, 1):
            s = (me + off) % N_DEV
            pltpu.make_async_remote_copy(
                src_ref=send_ref.at[pl.ds(0, 2)],
                dst_ref=recv_ref.at[s],
                send_sem=disp_send_sems.at[s],
                recv_sem=disp_recv_sems.at[s],
                device_id=(s,),
                device_id_type=pl.DeviceIdType.MESH,
            ).wait_recv()
            mlp(s)
            r = pltpu.make_async_remote_copy(
                src_ref=y_ref.at[s],
                dst_ref=ret_ref.at[pl.ds(2 * me, 2)],
                send_sem=ret_send_sems.at[s],
                recv_sem=ret_recv_sems.at[me],
                device_id=(s,),
                device_id_type=pl.DeviceIdType.MESH,
            )
            r.start()
            rets.append(r)

        out_ref[...] = lax.dot_general(
            p_chunk(me), ret_ref[pl.ds(2 * me, 2)].reshape(2 * CAP, D),
            (((0,), (0,)), ((), ())), preferred_element_type=jnp.float32)
        for off in (1, 2, 3):
            s = (me + off) % N_DEV
            pltpu.make_async_remote_copy(
                src_ref=y_ref.at[s],
                dst_ref=ret_ref.at[pl.ds(2 * s, 2)],
                send_sem=ret_send_sems.at[s],
                recv_sem=ret_recv_sems.at[s],
                device_id=(s,),
                device_id_type=pl.DeviceIdType.MESH,
            ).wait_recv()
            out_ref[...] = out_ref[...] + lax.dot_general(
                p_chunk(s), ret_ref[pl.ds(2 * s, 2)].reshape(2 * CAP, D),
                (((0,), (0,)), ((), ())), preferred_element_type=jnp.float32)

        for r in disp + rets:
            r.wait_send()

    return pl.pallas_call(
        body,
        out_shape=jax.ShapeDtypeStruct((T, D), jnp.float32),
        in_specs=[
            pl.BlockSpec(memory_space=pltpu.VMEM),
            pl.BlockSpec(memory_space=pltpu.VMEM),
            pl.BlockSpec(memory_space=pl.ANY),
            pl.BlockSpec(memory_space=pl.ANY),
        ],
        out_specs=pl.BlockSpec(memory_space=pltpu.VMEM),
        scratch_shapes=[
            pltpu.VMEM((N_EXP, CAP, D), jnp.bfloat16),
            pltpu.VMEM((N_DEV, 2, CAP, D), jnp.bfloat16),
            pltpu.VMEM((2, D, 2 * D), jnp.float32),
            pltpu.VMEM((2, 2 * D, D), jnp.float32),
            pltpu.SemaphoreType.DMA((N_DEV,)),
            pltpu.SemaphoreType.DMA((N_DEV,)),
            pltpu.SemaphoreType.DMA((N_DEV,)),
            pltpu.SemaphoreType.DMA((N_DEV,)),
            pltpu.SemaphoreType.DMA((2,)),
        ],
        compiler_params=pltpu.CompilerParams(
            collective_id=0, vmem_limit_bytes=110 * 1024 * 1024),
    )(x, slot2d, W1, W2)


def kernel(x, assign, W1, W2):
    T, _ = x.shape
    e = assign.astype(jnp.int32)

    oh = (e[:, None] == jnp.arange(N_EXP, dtype=jnp.int32)[None, :]).astype(
        jnp.int32)
    csum = jnp.cumsum(oh, axis=0)
    rank = jnp.sum(oh * csum, axis=1) - 1
    slot = jnp.where(rank < CAP, e * CAP + rank, S)

    return _moe_a2a_pallas(x, slot.reshape(1, T), W1, W2)
